# baseline (speedup 1.0000x reference)
import functools

import jax
import jax.numpy as jnp
from jax import lax
from jax.experimental import pallas as pl
from jax.experimental.pallas import tpu as pltpu
from jax.experimental.pallas import tpu_sc as plsc

CTX = 8192
DIM = 1024


@jax.jit
def _lookup(table):
    info = plsc.get_sparse_core_info()
    nw = info.num_cores * info.num_subcores
    rows_per_w = CTX // nw

    mesh = plsc.VectorSubcoreMesh(core_axis_name="c", subcore_axis_name="s")

    @functools.partial(
        pl.kernel,
        mesh=mesh,
        out_type=jax.ShapeDtypeStruct((CTX, DIM), jnp.float32),
    )
    def copy_kernel(table_hbm, out_hbm):
        wid = lax.axis_index("s") * info.num_cores + lax.axis_index("c")
        base = wid * rows_per_w
        pltpu.sync_copy(
            table_hbm.at[pl.ds(base, rows_per_w)],
            out_hbm.at[pl.ds(base, rows_per_w)],
        )

    return copy_kernel(table)


def kernel(table):
    return _lookup(table)


# speedup vs baseline: 24.4418x; 24.4418x over previous
import functools

import jax
import jax.numpy as jnp
from jax import lax
from jax.experimental import pallas as pl
from jax.experimental.pallas import tpu as pltpu
from jax.experimental.pallas import tpu_sc as plsc

CTX = 8192
DIM = 1024
ROWS_C = 32


@jax.jit
def _lookup(table):
    info = plsc.get_sparse_core_info()
    nw = info.num_cores * info.num_subcores
    rows_per_w = CTX // nw
    n_chunks = rows_per_w // ROWS_C

    mesh = plsc.VectorSubcoreMesh(core_axis_name="c", subcore_axis_name="s")

    @functools.partial(
        pl.kernel,
        mesh=mesh,
        out_type=jax.ShapeDtypeStruct((CTX, DIM), jnp.float32),
        scratch_types=[
            pltpu.VMEM((ROWS_C, DIM), jnp.float32),
            pltpu.VMEM((ROWS_C, DIM), jnp.float32),
            pltpu.SemaphoreType.DMA,
            pltpu.SemaphoreType.DMA,
            pltpu.SemaphoreType.DMA,
            pltpu.SemaphoreType.DMA,
        ],
    )
    def copy_kernel(table_hbm, out_hbm, buf0, buf1, rs0, rs1, ws0, ws1):
        wid = lax.axis_index("s") * info.num_cores + lax.axis_index("c")
        base = wid * rows_per_w
        bufs = (buf0, buf1)
        rsems = (rs0, rs1)
        wsems = (ws0, ws1)

        reads = [None, None]
        writes = [None, None]
        reads[0] = pltpu.async_copy(
            table_hbm.at[pl.ds(base, ROWS_C)], bufs[0], rsems[0]
        )
        for g in range(n_chunks):
            b = g % 2
            if g + 1 < n_chunks:
                nb = (g + 1) % 2
                if writes[nb] is not None:
                    writes[nb].wait()
                    writes[nb] = None
                reads[nb] = pltpu.async_copy(
                    table_hbm.at[pl.ds(base + (g + 1) * ROWS_C, ROWS_C)],
                    bufs[nb],
                    rsems[nb],
                )
            reads[b].wait()
            writes[b] = pltpu.async_copy(
                bufs[b],
                out_hbm.at[pl.ds(base + g * ROWS_C, ROWS_C)],
                wsems[b],
            )
        for w in writes:
            if w is not None:
                w.wait()

    return copy_kernel(table)


def kernel(table):
    return _lookup(table)


# speedup vs baseline: 24.4494x; 1.0003x over previous
import functools

import jax
import jax.numpy as jnp
from jax import lax
from jax.experimental import pallas as pl
from jax.experimental.pallas import tpu as pltpu
from jax.experimental.pallas import tpu_sc as plsc

CTX = 8192
DIM = 1024
ROWS_C = 40
NBUF = 3


@jax.jit
def _lookup(table):
    info = plsc.get_sparse_core_info()
    nw = info.num_cores * info.num_subcores
    rows_per_w = CTX // nw
    chunks = []
    r = 0
    while r < rows_per_w:
        c = min(ROWS_C, rows_per_w - r)
        chunks.append((r, c))
        r += c
    n_chunks = len(chunks)

    mesh = plsc.VectorSubcoreMesh(core_axis_name="c", subcore_axis_name="s")

    @functools.partial(
        pl.kernel,
        mesh=mesh,
        out_type=jax.ShapeDtypeStruct((CTX, DIM), jnp.float32),
        scratch_types=(
            [pltpu.VMEM((ROWS_C, DIM), jnp.float32)] * NBUF
            + [pltpu.SemaphoreType.DMA] * (2 * NBUF)
        ),
    )
    def copy_kernel(table_hbm, out_hbm, *scratch):
        bufs = scratch[:NBUF]
        rsems = scratch[NBUF : 2 * NBUF]
        wsems = scratch[2 * NBUF :]
        wid = lax.axis_index("s") * info.num_cores + lax.axis_index("c")
        base = wid * rows_per_w

        def start_read(g):
            off, cn = chunks[g]
            b = g % NBUF
            return pltpu.async_copy(
                table_hbm.at[pl.ds(base + off, cn)],
                bufs[b].at[pl.ds(0, cn)],
                rsems[b],
            )

        reads = [None] * NBUF
        writes = [None] * NBUF
        reads[0] = start_read(0)
        for g in range(n_chunks):
            b = g % NBUF
            off, cn = chunks[g]
            if g + 1 < n_chunks:
                nb = (g + 1) % NBUF
                if writes[nb] is not None:
                    writes[nb].wait()
                    writes[nb] = None
                reads[nb] = start_read(g + 1)
            reads[b].wait()
            writes[b] = pltpu.async_copy(
                bufs[b].at[pl.ds(0, cn)],
                out_hbm.at[pl.ds(base + off, cn)],
                wsems[b],
            )
        for w in writes:
            if w is not None:
                w.wait()

    return copy_kernel(table)


def kernel(table):
    return _lookup(table)
